# Initial kernel scaffold; baseline (speedup 1.0000x reference)
import jax
import jax.numpy as jnp
from jax.experimental import pallas as pl


def kernel(edge_attr, edge_index, ln_w, ln_b, W_ih0, W_hh0, b_ih0, b_hh0, W_ih1, W_hh1, b_ih1, b_hh1, W_ih2, W_hh2, b_ih2, b_hh2):
    raise NotImplementedError("write your pallas kernel here")


# speedup vs baseline: 5.6317x; 5.6317x over previous
import jax
import jax.numpy as jnp
from jax import lax
from jax.experimental import pallas as pl
from jax.experimental.pallas import tpu as pltpu
from jax.experimental.pallas import tpu_sc as plsc

_N = 512
_DIN = 16
_H = 128
_E = _N * _N
_B = 4096
_NB = _E // _B

_NW = 32
_EPW = _E // _NW
_CH = 128
_NCH = _EPW // _CH


def _stats_kernel(ea_ref, dstc_ref, stats_ref, cnt_ref):
    i = pl.program_id(0)

    @pl.when(i == 0)
    def _init():
        stats_ref[0, 0] = 0.0
        stats_ref[0, 1] = 0.0
        cnt_ref[...] = jnp.zeros_like(cnt_ref)

    ea = ea_ref[...]
    stats_ref[0, 0] += jnp.sum(ea)
    stats_ref[0, 1] += jnp.sum(ea * ea)
    one = jnp.bfloat16(1.0)
    zero = jnp.bfloat16(0.0)
    dst = jnp.broadcast_to(dstc_ref[0].astype(jnp.int16), (_B, _N))
    iota_b = jax.lax.broadcasted_iota(jnp.int16, (_B, _N), 1)
    ohd = jnp.where(iota_b == dst, one, zero)
    ones8 = jnp.full((8, _B), one)
    cnt_ref[...] += jnp.dot(ones8, ohd,
                            preferred_element_type=jnp.float32)


def _sigx_kernel(ea_ref, lnw_ref, lnb_ref, wcat_ref, ms_ref, x_ref):
    mu = ms_ref[0, 0]
    rstd = ms_ref[0, 1]
    y = (ea_ref[...] - mu) * rstd * lnw_ref[...] + lnb_ref[...]
    sig = jax.nn.sigmoid(y)
    x_ref[...] = jnp.dot(sig, wcat_ref[...],
                         preferred_element_type=jnp.float32)


def _gh_kernel(acc2_ref, inv_ref, whht_ref, bco_ref, g_ref, h_ref):
    h = (acc2_ref[0] + acc2_ref[1]) * inv_ref[...]
    h_ref[...] = h
    g_ref[...] = jnp.dot(h, whht_ref[...],
                         preferred_element_type=jnp.float32) + bco_ref[...]


def _tanh_kernel(x_ref, gg_ref, m_ref):
    m_ref[...] = jnp.tanh(x_ref[...] + gg_ref[...])


def _cbt_kernel(ofull_ref, oblk_ref, out_ref):
    o = ofull_ref[0]
    oi = oblk_ref[0]
    jc = min(128, _N)
    for j in range(_N // jc):
        oj = o[j * jc:(j + 1) * jc, :]
        d = jnp.sum(jnp.abs(oi[:, None, :] - oj[None, :, :]), axis=2)
        out_ref[0, :, j * jc:(j + 1) * jc] = d


def _sc_gather_body(g_hbm, idx_hbm, out_hbm, idx_v, rows_v, sem):
    wid = lax.axis_index("s") * 2 + lax.axis_index("c")
    pltpu.sync_copy(idx_hbm.at[wid], idx_v)

    def body(j, carry):
        base = wid * _EPW + j * _CH
        pltpu.async_copy(g_hbm.at[idx_v.at[j]], rows_v, sem).wait()
        pltpu.sync_copy(rows_v, out_hbm.at[pl.ds(base, _CH)])
        return carry

    lax.fori_loop(0, _NCH, body, 0)


def _sc_scatter_body(m_hbm, idx_hbm, zeros_hbm, out_hbm,
                     idx_v, rows_v, acc_s, sem):
    c = lax.axis_index("c")
    s = lax.axis_index("s")
    wid = s * 2 + c

    @pl.when(s == 0)
    def _zero():
        pltpu.sync_copy(zeros_hbm, acc_s)

    plsc.subcore_barrier()
    pltpu.sync_copy(idx_hbm.at[wid], idx_v)

    def body(j, carry):
        base = wid * _EPW + j * _CH
        pltpu.async_copy(m_hbm.at[pl.ds(base, _CH)], rows_v, sem).wait()
        pltpu.sync_copy(rows_v, acc_s.at[idx_v.at[j]], add=True)
        return carry

    lax.fori_loop(0, _NCH, body, 0)
    plsc.subcore_barrier()

    @pl.when(s == 0)
    def _out():
        pltpu.sync_copy(acc_s, out_hbm.at[c])


def kernel(edge_attr, edge_index, ln_w, ln_b, W_ih0, W_hh0, b_ih0, b_hh0,
           W_ih1, W_hh1, b_ih1, b_hh1, W_ih2, W_hh2, b_ih2, b_hh2):
    f32 = jnp.float32
    src = edge_index[0].astype(jnp.int32)
    dst = edge_index[1].astype(jnp.int32)
    dstc = dst.reshape(_NB, _B, 1)
    srcw = src.reshape(_NW, _NCH, _CH)
    dstw = dst.reshape(_NW, _NCH, _CH)

    wcat = jnp.zeros((_DIN, _H), f32)
    whht = jnp.zeros((_H, _H), f32)
    bco = jnp.zeros((1, _H), f32)
    off = 0
    for W_ih, W_hh, b_ih, b_hh in ((W_ih0, W_hh0, b_ih0, b_hh0),
                                   (W_ih1, W_hh1, b_ih1, b_hh1),
                                   (W_ih2, W_hh2, b_ih2, b_hh2)):
        h = W_ih.shape[0]
        wcat = wcat.at[:, off:off + h].set(W_ih.T)
        whht = whht.at[off:off + h, off:off + h].set(W_hh.T)
        bco = bco.at[0, off:off + h].set(b_ih + b_hh)
        off += h

    stats, cnt8 = pl.pallas_call(
        _stats_kernel,
        grid=(_NB,),
        in_specs=[
            pl.BlockSpec((_B, _DIN), lambda i: (i, 0)),
            pl.BlockSpec((1, _B, 1), lambda i: (i, 0, 0)),
        ],
        out_specs=[
            pl.BlockSpec(memory_space=pltpu.SMEM),
            pl.BlockSpec((8, _N), lambda i: (0, 0)),
        ],
        out_shape=[
            jax.ShapeDtypeStruct((1, 2), f32),
            jax.ShapeDtypeStruct((8, _N), f32),
        ],
        interpret=False,
    )(edge_attr, dstc)

    nelem = float(_E * _DIN)
    mu = stats[0, 0] / nelem
    var = stats[0, 1] / nelem - mu * mu
    rstd = jax.lax.rsqrt(var + 1e-5)
    musig = jnp.stack([mu, rstd]).reshape(1, 2)
    inv = (1.0 / jnp.maximum(cnt8[0], 1.0)).reshape(_N, 1)

    x_all = pl.pallas_call(
        _sigx_kernel,
        grid=(_NB,),
        in_specs=[
            pl.BlockSpec((_B, _DIN), lambda i: (i, 0)),
            pl.BlockSpec((_B, _DIN), lambda i: (i, 0)),
            pl.BlockSpec((_B, _DIN), lambda i: (i, 0)),
            pl.BlockSpec((_DIN, _H), lambda i: (0, 0)),
            pl.BlockSpec(memory_space=pltpu.SMEM),
        ],
        out_specs=pl.BlockSpec((_B, _H), lambda i: (i, 0)),
        out_shape=jax.ShapeDtypeStruct((_E, _H), f32),
        interpret=False,
    )(edge_attr, ln_w, ln_b, wcat, musig)

    gh = pl.pallas_call(
        _gh_kernel,
        out_shape=[
            jax.ShapeDtypeStruct((_N, _H), f32),
            jax.ShapeDtypeStruct((_N, _H), f32),
        ],
        interpret=False,
    )

    tanh_k = pl.pallas_call(
        _tanh_kernel,
        grid=(_NB,),
        in_specs=[
            pl.BlockSpec((_B, _H), lambda i: (i, 0)),
            pl.BlockSpec((_B, _H), lambda i: (i, 0)),
        ],
        out_specs=pl.BlockSpec((_B, _H), lambda i: (i, 0)),
        out_shape=jax.ShapeDtypeStruct((_E, _H), f32),
        interpret=False,
    )

    mesh = plsc.VectorSubcoreMesh(core_axis_name="c", subcore_axis_name="s")
    sc_gather = pl.kernel(
        _sc_gather_body,
        mesh=mesh,
        out_type=jax.ShapeDtypeStruct((_E, _H), f32),
        scratch_types=[
            pltpu.VMEM((_NCH, _CH), jnp.int32),
            pltpu.VMEM((_CH, _H), f32),
            pltpu.SemaphoreType.DMA,
        ],
    )
    sc_scatter = pl.kernel(
        _sc_scatter_body,
        mesh=mesh,
        out_type=jax.ShapeDtypeStruct((2, _N, _H), f32),
        scratch_types=[
            pltpu.VMEM((_NCH, _CH), jnp.int32),
            pltpu.VMEM((_CH, _H), f32),
            pltpu.VMEM_SHARED((_N, _H), f32),
            pltpu.SemaphoreType.DMA,
        ],
    )

    zeros_nh = jnp.zeros((_N, _H), f32)
    acc2 = jnp.zeros((2, _N, _H), f32)
    hs = []
    for _ in range(3):
        g, h_cur = gh(acc2, inv, whht, bco)
        hs.append(h_cur)
        gg = sc_gather(g, srcw)
        m = tanh_k(x_all, gg)
        acc2 = sc_scatter(m, dstw, zeros_nh)
    _, h3 = gh(acc2, inv, whht, bco)

    houts = jnp.stack([h3, hs[2]])

    rb = 64
    cbt = pl.pallas_call(
        _cbt_kernel,
        grid=(2, _N // rb),
        in_specs=[
            pl.BlockSpec((1, _N, _H), lambda t, i: (t, 0, 0)),
            pl.BlockSpec((1, rb, _H), lambda t, i: (t, i, 0)),
        ],
        out_specs=pl.BlockSpec((1, rb, _N), lambda t, i: (t, i, 0)),
        out_shape=jax.ShapeDtypeStruct((2, _N, _N), f32),
        interpret=False,
    )(houts, houts)

    return cbt


# speedup vs baseline: 5.8004x; 1.0300x over previous
import jax
import jax.numpy as jnp
from jax import lax
from jax.experimental import pallas as pl
from jax.experimental.pallas import tpu as pltpu
from jax.experimental.pallas import tpu_sc as plsc

_N = 512
_DIN = 16
_H = 128
_E = _N * _N
_B = 4096
_NB = _E // _B

_NW = 32
_EPW = _E // _NW
_CH = 128
_NCH = _EPW // _CH


def _stats_kernel(ea_ref, dstc_ref, stats_ref, cnt_ref):
    i = pl.program_id(0)

    @pl.when(i == 0)
    def _init():
        stats_ref[0, 0] = 0.0
        stats_ref[0, 1] = 0.0
        cnt_ref[...] = jnp.zeros_like(cnt_ref)

    ea = ea_ref[...]
    stats_ref[0, 0] += jnp.sum(ea)
    stats_ref[0, 1] += jnp.sum(ea * ea)
    one = jnp.bfloat16(1.0)
    zero = jnp.bfloat16(0.0)
    dst = jnp.broadcast_to(dstc_ref[0].astype(jnp.int16), (_B, _N))
    iota_b = jax.lax.broadcasted_iota(jnp.int16, (_B, _N), 1)
    ohd = jnp.where(iota_b == dst, one, zero)
    ones8 = jnp.full((8, _B), one)
    cnt_ref[...] += jnp.dot(ones8, ohd,
                            preferred_element_type=jnp.float32)


def _sigx_kernel(ea_ref, lnw_ref, lnb_ref, wcat_ref, ms_ref, x_ref):
    mu = ms_ref[0, 0]
    rstd = ms_ref[0, 1]
    y = (ea_ref[...] - mu) * rstd * lnw_ref[...] + lnb_ref[...]
    sig = jax.nn.sigmoid(y)
    x_ref[...] = jnp.dot(sig, wcat_ref[...],
                         preferred_element_type=jnp.float32)


def _gh_kernel(acc2_ref, inv_ref, whht_ref, bco_ref, g_ref, h_ref):
    h = (acc2_ref[0] + acc2_ref[1]) * inv_ref[...]
    h_ref[...] = h
    g_ref[...] = jnp.dot(h, whht_ref[...],
                         preferred_element_type=jnp.float32) + bco_ref[...]


def _tanh_kernel(x_ref, gg_ref, m_ref):
    m_ref[...] = jnp.tanh(x_ref[...] + gg_ref[...])


def _cbt_kernel(ofull_ref, oblk_ref, out_ref):
    o = ofull_ref[0]
    oi = oblk_ref[0]
    jc = min(128, _N)
    for j in range(_N // jc):
        oj = o[j * jc:(j + 1) * jc, :]
        d = jnp.sum(jnp.abs(oi[:, None, :] - oj[None, :, :]), axis=2)
        out_ref[0, :, j * jc:(j + 1) * jc] = d


def _sc_gather_body(g_hbm, idx_hbm, out_hbm, idx_v, rows_a, rows_b,
                    sem_a, sem_b):
    wid = lax.axis_index("s") * 2 + lax.axis_index("c")
    pltpu.sync_copy(idx_hbm.at[wid], idx_v)

    def body(j, carry):
        b0 = wid * _EPW + (2 * j) * _CH
        b1 = b0 + _CH
        cp0 = pltpu.async_copy(g_hbm.at[idx_v.at[2 * j]], rows_a, sem_a)
        cp1 = pltpu.async_copy(g_hbm.at[idx_v.at[2 * j + 1]], rows_b, sem_b)
        cp0.wait()
        pltpu.sync_copy(rows_a, out_hbm.at[pl.ds(b0, _CH)])
        cp1.wait()
        pltpu.sync_copy(rows_b, out_hbm.at[pl.ds(b1, _CH)])
        return carry

    lax.fori_loop(0, _NCH // 2, body, 0)


def _sc_scatter_body(m_hbm, idx_hbm, zeros_hbm, out_hbm,
                     idx_v, rows_a, rows_b, acc_s, sem_a, sem_b):
    c = lax.axis_index("c")
    s = lax.axis_index("s")
    wid = s * 2 + c

    @pl.when(s == 0)
    def _zero():
        pltpu.sync_copy(zeros_hbm, acc_s)

    plsc.subcore_barrier()
    pltpu.sync_copy(idx_hbm.at[wid], idx_v)

    def body(j, carry):
        b0 = wid * _EPW + (2 * j) * _CH
        b1 = b0 + _CH
        cp0 = pltpu.async_copy(m_hbm.at[pl.ds(b0, _CH)], rows_a, sem_a)
        cp1 = pltpu.async_copy(m_hbm.at[pl.ds(b1, _CH)], rows_b, sem_b)
        cp0.wait()
        pltpu.sync_copy(rows_a, acc_s.at[idx_v.at[2 * j]], add=True)
        cp1.wait()
        pltpu.sync_copy(rows_b, acc_s.at[idx_v.at[2 * j + 1]], add=True)
        return carry

    lax.fori_loop(0, _NCH // 2, body, 0)
    plsc.subcore_barrier()

    @pl.when(s == 0)
    def _out():
        pltpu.sync_copy(acc_s, out_hbm.at[c])


def kernel(edge_attr, edge_index, ln_w, ln_b, W_ih0, W_hh0, b_ih0, b_hh0,
           W_ih1, W_hh1, b_ih1, b_hh1, W_ih2, W_hh2, b_ih2, b_hh2):
    f32 = jnp.float32
    src = edge_index[0].astype(jnp.int32)
    dst = edge_index[1].astype(jnp.int32)
    dstc = dst.reshape(_NB, _B, 1)
    srcw = src.reshape(_NW, _NCH, _CH)
    dstw = dst.reshape(_NW, _NCH, _CH)

    wcat = jnp.zeros((_DIN, _H), f32)
    whht = jnp.zeros((_H, _H), f32)
    bco = jnp.zeros((1, _H), f32)
    off = 0
    for W_ih, W_hh, b_ih, b_hh in ((W_ih0, W_hh0, b_ih0, b_hh0),
                                   (W_ih1, W_hh1, b_ih1, b_hh1),
                                   (W_ih2, W_hh2, b_ih2, b_hh2)):
        h = W_ih.shape[0]
        wcat = wcat.at[:, off:off + h].set(W_ih.T)
        whht = whht.at[off:off + h, off:off + h].set(W_hh.T)
        bco = bco.at[0, off:off + h].set(b_ih + b_hh)
        off += h

    stats, cnt8 = pl.pallas_call(
        _stats_kernel,
        grid=(_NB,),
        in_specs=[
            pl.BlockSpec((_B, _DIN), lambda i: (i, 0)),
            pl.BlockSpec((1, _B, 1), lambda i: (i, 0, 0)),
        ],
        out_specs=[
            pl.BlockSpec(memory_space=pltpu.SMEM),
            pl.BlockSpec((8, _N), lambda i: (0, 0)),
        ],
        out_shape=[
            jax.ShapeDtypeStruct((1, 2), f32),
            jax.ShapeDtypeStruct((8, _N), f32),
        ],
        interpret=False,
    )(edge_attr, dstc)

    nelem = float(_E * _DIN)
    mu = stats[0, 0] / nelem
    var = stats[0, 1] / nelem - mu * mu
    rstd = jax.lax.rsqrt(var + 1e-5)
    musig = jnp.stack([mu, rstd]).reshape(1, 2)
    inv = (1.0 / jnp.maximum(cnt8[0], 1.0)).reshape(_N, 1)

    x_all = pl.pallas_call(
        _sigx_kernel,
        grid=(_NB,),
        in_specs=[
            pl.BlockSpec((_B, _DIN), lambda i: (i, 0)),
            pl.BlockSpec((_B, _DIN), lambda i: (i, 0)),
            pl.BlockSpec((_B, _DIN), lambda i: (i, 0)),
            pl.BlockSpec((_DIN, _H), lambda i: (0, 0)),
            pl.BlockSpec(memory_space=pltpu.SMEM),
        ],
        out_specs=pl.BlockSpec((_B, _H), lambda i: (i, 0)),
        out_shape=jax.ShapeDtypeStruct((_E, _H), f32),
        interpret=False,
    )(edge_attr, ln_w, ln_b, wcat, musig)

    gh = pl.pallas_call(
        _gh_kernel,
        out_shape=[
            jax.ShapeDtypeStruct((_N, _H), f32),
            jax.ShapeDtypeStruct((_N, _H), f32),
        ],
        interpret=False,
    )

    tanh_k = pl.pallas_call(
        _tanh_kernel,
        grid=(_NB,),
        in_specs=[
            pl.BlockSpec((_B, _H), lambda i: (i, 0)),
            pl.BlockSpec((_B, _H), lambda i: (i, 0)),
        ],
        out_specs=pl.BlockSpec((_B, _H), lambda i: (i, 0)),
        out_shape=jax.ShapeDtypeStruct((_E, _H), f32),
        interpret=False,
    )

    mesh = plsc.VectorSubcoreMesh(core_axis_name="c", subcore_axis_name="s")
    sc_gather = pl.kernel(
        _sc_gather_body,
        mesh=mesh,
        out_type=jax.ShapeDtypeStruct((_E, _H), f32),
        scratch_types=[
            pltpu.VMEM((_NCH, _CH), jnp.int32),
            pltpu.VMEM((_CH, _H), f32),
            pltpu.VMEM((_CH, _H), f32),
            pltpu.SemaphoreType.DMA,
            pltpu.SemaphoreType.DMA,
        ],
    )
    sc_scatter = pl.kernel(
        _sc_scatter_body,
        mesh=mesh,
        out_type=jax.ShapeDtypeStruct((2, _N, _H), f32),
        scratch_types=[
            pltpu.VMEM((_NCH, _CH), jnp.int32),
            pltpu.VMEM((_CH, _H), f32),
            pltpu.VMEM((_CH, _H), f32),
            pltpu.VMEM_SHARED((_N, _H), f32),
            pltpu.SemaphoreType.DMA,
            pltpu.SemaphoreType.DMA,
        ],
    )

    zeros_nh = jnp.zeros((_N, _H), f32)
    acc2 = jnp.zeros((2, _N, _H), f32)
    hs = []
    for _ in range(3):
        g, h_cur = gh(acc2, inv, whht, bco)
        hs.append(h_cur)
        gg = sc_gather(g, srcw)
        m = tanh_k(x_all, gg)
        acc2 = sc_scatter(m, dstw, zeros_nh)
    _, h3 = gh(acc2, inv, whht, bco)

    houts = jnp.stack([h3, hs[2]])

    rb = 64
    cbt = pl.pallas_call(
        _cbt_kernel,
        grid=(2, _N // rb),
        in_specs=[
            pl.BlockSpec((1, _N, _H), lambda t, i: (t, 0, 0)),
            pl.BlockSpec((1, rb, _H), lambda t, i: (t, i, 0)),
        ],
        out_specs=pl.BlockSpec((1, rb, _N), lambda t, i: (t, i, 0)),
        out_shape=jax.ShapeDtypeStruct((2, _N, _N), f32),
        interpret=False,
    )(houts, houts)

    return cbt


# speedup vs baseline: 10.8961x; 1.8785x over previous
import jax
import jax.numpy as jnp
from jax.experimental import pallas as pl
from jax.experimental.pallas import tpu as pltpu

_N = 512
_DIN = 16
_H = 128
_E = _N * _N
_B = 4096
_NB = _E // _B


def _stats_kernel(ea_ref, stats_ref):
    i = pl.program_id(0)

    @pl.when(i == 0)
    def _init():
        stats_ref[0, 0] = 0.0
        stats_ref[0, 1] = 0.0

    ea = ea_ref[...]
    stats_ref[0, 0] += jnp.sum(ea)
    stats_ref[0, 1] += jnp.sum(ea * ea)


def _sigmoid_kernel(ea_ref, lnw_ref, lnb_ref, ms_ref, out_ref):
    mu = ms_ref[0, 0]
    rstd = ms_ref[0, 1]
    y = (ea_ref[...] - mu) * rstd * lnw_ref[...] + lnb_ref[...]
    out_ref[...] = jax.nn.sigmoid(y)


def _step_kernel(sigt_ref, srcr_ref, dstc_ref, ht_ref, whh2_ref, bcatt_ref,
                 wcatt_ref, htout_ref, acct_ref, gbft_ref):
    i = pl.program_id(0)

    @pl.when(i == 0)
    def _init():
        gt = jnp.dot(whh2_ref[...], ht_ref[...],
                     preferred_element_type=jnp.float32) + bcatt_ref[...]
        gbft_ref[...] = gt.astype(jnp.bfloat16)
        acct_ref[...] = jnp.zeros_like(acct_ref)

    one = jnp.bfloat16(1.0)
    zero = jnp.bfloat16(0.0)

    xt = jnp.dot(wcatt_ref[...], sigt_ref[...],
                 preferred_element_type=jnp.float32)
    src = jnp.broadcast_to(srcr_ref[0], (_N, _B))
    iota_n = jax.lax.broadcasted_iota(jnp.int32, (_N, _B), 0)
    ohst = jnp.where(iota_n == src, 1.0, 0.0).astype(jnp.bfloat16)
    gst = jnp.dot(gbft_ref[...], ohst,
                  preferred_element_type=jnp.float32)
    mt = jnp.tanh(xt + gst)
    maug = jnp.concatenate(
        [mt.astype(jnp.bfloat16), jnp.full((8, _B), one)], axis=0)
    dst = jnp.broadcast_to(dstc_ref[0].astype(jnp.int16), (_B, _N))
    iota_b = jax.lax.broadcasted_iota(jnp.int16, (_B, _N), 1)
    ohdt = jnp.where(iota_b == dst, one, zero)
    acct_ref[...] += jnp.dot(maug, ohdt,
                             preferred_element_type=jnp.float32)

    @pl.when(i == _NB - 1)
    def _fin():
        cnt = acct_ref[_H:_H + 1, :]
        htout_ref[...] = acct_ref[:_H, :] / jnp.maximum(cnt, 1.0)


def _cbt_kernel(ofull_ref, oblk_ref, out_ref):
    o = ofull_ref[0]
    oi = oblk_ref[0]
    jc = min(128, _N)
    for j in range(_N // jc):
        oj = o[j * jc:(j + 1) * jc, :]
        d = jnp.sum(jnp.abs(oi[:, None, :] - oj[None, :, :]), axis=2)
        out_ref[0, :, j * jc:(j + 1) * jc] = d


def kernel(edge_attr, edge_index, ln_w, ln_b, W_ih0, W_hh0, b_ih0, b_hh0,
           W_ih1, W_hh1, b_ih1, b_hh1, W_ih2, W_hh2, b_ih2, b_hh2):
    f32 = jnp.float32
    src = edge_index[0].astype(jnp.int32)
    dst = edge_index[1].astype(jnp.int32)
    srcr = src.reshape(_NB, 1, _B)
    dstc = dst.reshape(_NB, _B, 1)

    wcatt = jnp.zeros((_H, _DIN), f32)
    whh2 = jnp.zeros((_H, _H), f32)
    bcatt = jnp.zeros((_H, 1), f32)
    off = 0
    for W_ih, W_hh, b_ih, b_hh in ((W_ih0, W_hh0, b_ih0, b_hh0),
                                   (W_ih1, W_hh1, b_ih1, b_hh1),
                                   (W_ih2, W_hh2, b_ih2, b_hh2)):
        h = W_ih.shape[0]
        wcatt = wcatt.at[off:off + h, :].set(W_ih)
        whh2 = whh2.at[off:off + h, off:off + h].set(W_hh)
        bcatt = bcatt.at[off:off + h, 0].set(b_ih + b_hh)
        off += h

    stats = pl.pallas_call(
        _stats_kernel,
        grid=(_NB,),
        in_specs=[pl.BlockSpec((_B, _DIN), lambda i: (i, 0))],
        out_specs=pl.BlockSpec(memory_space=pltpu.SMEM),
        out_shape=jax.ShapeDtypeStruct((1, 2), f32),
        interpret=False,
    )(edge_attr)

    nelem = float(_E * _DIN)
    mu = stats[0, 0] / nelem
    var = stats[0, 1] / nelem - mu * mu
    rstd = jax.lax.rsqrt(var + 1e-5)
    musig = jnp.stack([mu, rstd]).reshape(1, 2)

    sig = pl.pallas_call(
        _sigmoid_kernel,
        grid=(_NB,),
        in_specs=[
            pl.BlockSpec((_B, _DIN), lambda i: (i, 0)),
            pl.BlockSpec((_B, _DIN), lambda i: (i, 0)),
            pl.BlockSpec((_B, _DIN), lambda i: (i, 0)),
            pl.BlockSpec(memory_space=pltpu.SMEM),
        ],
        out_specs=pl.BlockSpec((_B, _DIN), lambda i: (i, 0)),
        out_shape=jax.ShapeDtypeStruct((_E, _DIN), f32),
        interpret=False,
    )(edge_attr, ln_w, ln_b, musig)
    sigt = sig.T

    step = pl.pallas_call(
        _step_kernel,
        grid=(_NB,),
        in_specs=[
            pl.BlockSpec((_DIN, _B), lambda i: (0, i)),
            pl.BlockSpec((1, 1, _B), lambda i: (i, 0, 0)),
            pl.BlockSpec((1, _B, 1), lambda i: (i, 0, 0)),
            pl.BlockSpec((_H, _N), lambda i: (0, 0)),
            pl.BlockSpec((_H, _H), lambda i: (0, 0)),
            pl.BlockSpec((_H, 1), lambda i: (0, 0)),
            pl.BlockSpec((_H, _DIN), lambda i: (0, 0)),
        ],
        out_specs=pl.BlockSpec((_H, _N), lambda i: (0, 0)),
        out_shape=jax.ShapeDtypeStruct((_H, _N), f32),
        scratch_shapes=[
            pltpu.VMEM((_H + 8, _N), f32),
            pltpu.VMEM((_H, _N), jnp.bfloat16),
        ],
        interpret=False,
    )

    ht = jnp.zeros((_H, _N), f32)
    ht1 = step(sigt, srcr, dstc, ht, whh2, bcatt, wcatt)
    ht2 = step(sigt, srcr, dstc, ht1, whh2, bcatt, wcatt)
    ht3 = step(sigt, srcr, dstc, ht2, whh2, bcatt, wcatt)

    houts = jnp.stack([ht3.T, ht2.T])

    rb = 64
    cbt = pl.pallas_call(
        _cbt_kernel,
        grid=(2, _N // rb),
        in_specs=[
            pl.BlockSpec((1, _N, _H), lambda t, i: (t, 0, 0)),
            pl.BlockSpec((1, rb, _H), lambda t, i: (t, i, 0)),
        ],
        out_specs=pl.BlockSpec((1, rb, _N), lambda t, i: (t, i, 0)),
        out_shape=jax.ShapeDtypeStruct((2, _N, _N), f32),
        interpret=False,
    )(houts, houts)

    return cbt


# speedup vs baseline: 12.1211x; 1.1124x over previous
import jax
import jax.numpy as jnp
from jax.experimental import pallas as pl
from jax.experimental.pallas import tpu as pltpu

_N = 512
_DIN = 16
_H = 128
_E = _N * _N
_B = 4096
_NB = _E // _B


def _stats_kernel(ea_ref, stats_ref):
    i = pl.program_id(0)

    @pl.when(i == 0)
    def _init():
        stats_ref[0, 0] = 0.0
        stats_ref[0, 1] = 0.0

    ea = ea_ref[...]
    stats_ref[0, 0] += jnp.sum(ea)
    stats_ref[0, 1] += jnp.sum(ea * ea)


def _sigmoid_kernel(ea_ref, lnw_ref, lnb_ref, ms_ref, out_ref):
    mu = ms_ref[0, 0]
    rstd = ms_ref[0, 1]
    y = (ea_ref[...] - mu) * rstd * lnw_ref[...] + lnb_ref[...]
    out_ref[...] = jax.nn.sigmoid(y).T


def _step_kernel(sigt_ref, srcr_ref, dstc_ref, ht_ref, whh2_ref, bcatt_ref,
                 wcatt_ref, htout_ref, acct_ref, gbft_ref):
    i = pl.program_id(0)

    @pl.when(i == 0)
    def _init():
        gt = jnp.dot(whh2_ref[...], ht_ref[...],
                     preferred_element_type=jnp.float32) + bcatt_ref[...]
        gbft_ref[...] = gt.astype(jnp.bfloat16)
        acct_ref[...] = jnp.zeros_like(acct_ref)

    one = jnp.bfloat16(1.0)
    zero = jnp.bfloat16(0.0)

    xt = jnp.dot(wcatt_ref[...], sigt_ref[...],
                 preferred_element_type=jnp.float32)
    src = jnp.broadcast_to(srcr_ref[0], (_N, _B))
    iota_n = jax.lax.broadcasted_iota(jnp.int32, (_N, _B), 0)
    ohst = jnp.where(iota_n == src, 1.0, 0.0).astype(jnp.bfloat16)
    gst = jnp.dot(gbft_ref[...], ohst,
                  preferred_element_type=jnp.float32)
    mt = jnp.tanh(xt + gst)
    maug = jnp.concatenate(
        [mt.astype(jnp.bfloat16), jnp.full((8, _B), one)], axis=0)
    dst = jnp.broadcast_to(dstc_ref[0].astype(jnp.int16), (_B, _N))
    iota_b = jax.lax.broadcasted_iota(jnp.int16, (_B, _N), 1)
    ohdt = jnp.where(iota_b == dst, one, zero)
    acct_ref[...] += jnp.dot(maug, ohdt,
                             preferred_element_type=jnp.float32)

    @pl.when(i == _NB - 1)
    def _fin():
        cnt = acct_ref[_H:_H + 1, :]
        htout_ref[...] = acct_ref[:_H, :] / jnp.maximum(cnt, 1.0)


def _cbt_kernel(ofull_ref, oblk_ref, out_ref):
    o = ofull_ref[0]
    oi = oblk_ref[0]
    jc = min(128, _N)
    for j in range(_N // jc):
        oj = o[j * jc:(j + 1) * jc, :]
        d = jnp.sum(jnp.abs(oi[:, None, :] - oj[None, :, :]), axis=2)
        out_ref[0, :, j * jc:(j + 1) * jc] = d


def kernel(edge_attr, edge_index, ln_w, ln_b, W_ih0, W_hh0, b_ih0, b_hh0,
           W_ih1, W_hh1, b_ih1, b_hh1, W_ih2, W_hh2, b_ih2, b_hh2):
    f32 = jnp.float32
    src = edge_index[0].astype(jnp.int32)
    dst = edge_index[1].astype(jnp.int32)
    srcr = src.reshape(_NB, 1, _B)
    dstc = dst.reshape(_NB, _B, 1)

    wcatt = jnp.zeros((_H, _DIN), f32)
    whh2 = jnp.zeros((_H, _H), f32)
    bcatt = jnp.zeros((_H, 1), f32)
    off = 0
    for W_ih, W_hh, b_ih, b_hh in ((W_ih0, W_hh0, b_ih0, b_hh0),
                                   (W_ih1, W_hh1, b_ih1, b_hh1),
                                   (W_ih2, W_hh2, b_ih2, b_hh2)):
        h = W_ih.shape[0]
        wcatt = wcatt.at[off:off + h, :].set(W_ih)
        whh2 = whh2.at[off:off + h, off:off + h].set(W_hh)
        bcatt = bcatt.at[off:off + h, 0].set(b_ih + b_hh)
        off += h

    stats = pl.pallas_call(
        _stats_kernel,
        grid=(_NB,),
        in_specs=[pl.BlockSpec((_B, _DIN), lambda i: (i, 0))],
        out_specs=pl.BlockSpec(memory_space=pltpu.SMEM),
        out_shape=jax.ShapeDtypeStruct((1, 2), f32),
        interpret=False,
    )(edge_attr)

    nelem = float(_E * _DIN)
    mu = stats[0, 0] / nelem
    var = stats[0, 1] / nelem - mu * mu
    rstd = jax.lax.rsqrt(var + 1e-5)
    musig = jnp.stack([mu, rstd]).reshape(1, 2)

    sigt = pl.pallas_call(
        _sigmoid_kernel,
        grid=(_NB,),
        in_specs=[
            pl.BlockSpec((_B, _DIN), lambda i: (i, 0)),
            pl.BlockSpec((_B, _DIN), lambda i: (i, 0)),
            pl.BlockSpec((_B, _DIN), lambda i: (i, 0)),
            pl.BlockSpec(memory_space=pltpu.SMEM),
        ],
        out_specs=pl.BlockSpec((_DIN, _B), lambda i: (0, i)),
        out_shape=jax.ShapeDtypeStruct((_DIN, _E), f32),
        interpret=False,
    )(edge_attr, ln_w, ln_b, musig)

    step = pl.pallas_call(
        _step_kernel,
        grid=(_NB,),
        in_specs=[
            pl.BlockSpec((_DIN, _B), lambda i: (0, i)),
            pl.BlockSpec((1, 1, _B), lambda i: (i, 0, 0)),
            pl.BlockSpec((1, _B, 1), lambda i: (i, 0, 0)),
            pl.BlockSpec((_H, _N), lambda i: (0, 0)),
            pl.BlockSpec((_H, _H), lambda i: (0, 0)),
            pl.BlockSpec((_H, 1), lambda i: (0, 0)),
            pl.BlockSpec((_H, _DIN), lambda i: (0, 0)),
        ],
        out_specs=pl.BlockSpec((_H, _N), lambda i: (0, 0)),
        out_shape=jax.ShapeDtypeStruct((_H, _N), f32),
        scratch_shapes=[
            pltpu.VMEM((_H + 8, _N), f32),
            pltpu.VMEM((_H, _N), jnp.bfloat16),
        ],
        interpret=False,
    )

    ht = jnp.zeros((_H, _N), f32)
    ht1 = step(sigt, srcr, dstc, ht, whh2, bcatt, wcatt)
    ht2 = step(sigt, srcr, dstc, ht1, whh2, bcatt, wcatt)
    ht3 = step(sigt, srcr, dstc, ht2, whh2, bcatt, wcatt)

    houts = jnp.stack([ht3.T, ht2.T])

    rb = 64
    cbt = pl.pallas_call(
        _cbt_kernel,
        grid=(2, _N // rb),
        in_specs=[
            pl.BlockSpec((1, _N, _H), lambda t, i: (t, 0, 0)),
            pl.BlockSpec((1, rb, _H), lambda t, i: (t, i, 0)),
        ],
        out_specs=pl.BlockSpec((1, rb, _N), lambda t, i: (t, i, 0)),
        out_shape=jax.ShapeDtypeStruct((2, _N, _N), f32),
        interpret=False,
    )(houts, houts)

    return cbt


# speedup vs baseline: 18.2055x; 1.5020x over previous
import jax
import jax.numpy as jnp
from jax.experimental import pallas as pl
from jax.experimental.pallas import tpu as pltpu

_N = 512
_DIN = 16
_H = 128
_E = _N * _N
_B = 4096
_NB = _E // _B


def _stats_kernel(ea_ref, stats_ref):
    i = pl.program_id(0)

    @pl.when(i == 0)
    def _init():
        stats_ref[0, 0] = 0.0
        stats_ref[0, 1] = 0.0

    ea = ea_ref[...]
    stats_ref[0, 0] += jnp.sum(ea)
    stats_ref[0, 1] += jnp.sum(ea * ea)


def _sigmoid_kernel(ea_ref, lnw_ref, lnb_ref, ms_ref, out_ref):
    mu = ms_ref[0, 0]
    rstd = ms_ref[0, 1]
    y = (ea_ref[...] - mu) * rstd * lnw_ref[...] + lnb_ref[...]
    out_ref[...] = jax.nn.sigmoid(y)


def _step_kernel(sigt_ref, srcr_ref, dstc_ref, ht_ref, whh2_ref, bcatt_ref,
                 wcatt_ref, htout_ref, acct_ref, gbft_ref):
    i = pl.program_id(0)

    @pl.when(i == 0)
    def _init():
        gt = jnp.dot(whh2_ref[...], ht_ref[...],
                     preferred_element_type=jnp.float32) + bcatt_ref[...]
        gbft_ref[...] = gt.astype(jnp.bfloat16)
        acct_ref[...] = jnp.zeros_like(acct_ref)

    one = jnp.bfloat16(1.0)
    zero = jnp.bfloat16(0.0)

    xt = jnp.dot(wcatt_ref[...], sigt_ref[...],
                 preferred_element_type=jnp.float32)
    src = jnp.broadcast_to(srcr_ref[0], (_N, _B))
    iota_n = jax.lax.broadcasted_iota(jnp.int32, (_N, _B), 0)
    ohst = jnp.where(iota_n == src, 1.0, 0.0).astype(jnp.bfloat16)
    gst = jnp.dot(gbft_ref[...], ohst,
                  preferred_element_type=jnp.float32)
    mt = jnp.tanh(xt + gst)
    maug = jnp.concatenate(
        [mt.astype(jnp.bfloat16), jnp.full((8, _B), one)], axis=0)
    dst = jnp.broadcast_to(dstc_ref[0].astype(jnp.int16), (_B, _N))
    iota_b = jax.lax.broadcasted_iota(jnp.int16, (_B, _N), 1)
    ohdt = jnp.where(iota_b == dst, one, zero)
    acct_ref[...] += jnp.dot(maug, ohdt,
                             preferred_element_type=jnp.float32)

    @pl.when(i == _NB - 1)
    def _fin():
        cnt = acct_ref[_H:_H + 1, :]
        htout_ref[...] = acct_ref[:_H, :] / jnp.maximum(cnt, 1.0)


def _cbt_kernel(ofull_ref, oblk_ref, out_ref):
    o = ofull_ref[0]
    oi = oblk_ref[0]
    jc = min(128, _N)
    for j in range(_N // jc):
        oj = o[j * jc:(j + 1) * jc, :]
        d = jnp.sum(jnp.abs(oi[:, None, :] - oj[None, :, :]), axis=2)
        out_ref[0, :, j * jc:(j + 1) * jc] = d


def kernel(edge_attr, edge_index, ln_w, ln_b, W_ih0, W_hh0, b_ih0, b_hh0,
           W_ih1, W_hh1, b_ih1, b_hh1, W_ih2, W_hh2, b_ih2, b_hh2):
    f32 = jnp.float32
    src = edge_index[0].astype(jnp.int32)
    dst = edge_index[1].astype(jnp.int32)
    srcr = src.reshape(_NB, 1, _B)
    dstc = dst.reshape(_NB, _B, 1)
    eat = edge_attr.T
    lnwt = ln_w.T
    lnbt = ln_b.T

    wcatt = jnp.zeros((_H, _DIN), f32)
    whh2 = jnp.zeros((_H, _H), f32)
    bcatt = jnp.zeros((_H, 1), f32)
    off = 0
    for W_ih, W_hh, b_ih, b_hh in ((W_ih0, W_hh0, b_ih0, b_hh0),
                                   (W_ih1, W_hh1, b_ih1, b_hh1),
                                   (W_ih2, W_hh2, b_ih2, b_hh2)):
        h = W_ih.shape[0]
        wcatt = wcatt.at[off:off + h, :].set(W_ih)
        whh2 = whh2.at[off:off + h, off:off + h].set(W_hh)
        bcatt = bcatt.at[off:off + h, 0].set(b_ih + b_hh)
        off += h

    stats = pl.pallas_call(
        _stats_kernel,
        grid=(_NB,),
        in_specs=[pl.BlockSpec((_DIN, _B), lambda i: (0, i))],
        out_specs=pl.BlockSpec(memory_space=pltpu.SMEM),
        out_shape=jax.ShapeDtypeStruct((1, 2), f32),
        interpret=False,
    )(eat)

    nelem = float(_E * _DIN)
    mu = stats[0, 0] / nelem
    var = stats[0, 1] / nelem - mu * mu
    rstd = jax.lax.rsqrt(var + 1e-5)
    musig = jnp.stack([mu, rstd]).reshape(1, 2)

    sigt = pl.pallas_call(
        _sigmoid_kernel,
        grid=(_NB,),
        in_specs=[
            pl.BlockSpec((_DIN, _B), lambda i: (0, i)),
            pl.BlockSpec((_DIN, _B), lambda i: (0, i)),
            pl.BlockSpec((_DIN, _B), lambda i: (0, i)),
            pl.BlockSpec(memory_space=pltpu.SMEM),
        ],
        out_specs=pl.BlockSpec((_DIN, _B), lambda i: (0, i)),
        out_shape=jax.ShapeDtypeStruct((_DIN, _E), f32),
        interpret=False,
    )(eat, lnwt, lnbt, musig)

    step = pl.pallas_call(
        _step_kernel,
        grid=(_NB,),
        in_specs=[
            pl.BlockSpec((_DIN, _B), lambda i: (0, i)),
            pl.BlockSpec((1, 1, _B), lambda i: (i, 0, 0)),
            pl.BlockSpec((1, _B, 1), lambda i: (i, 0, 0)),
            pl.BlockSpec((_H, _N), lambda i: (0, 0)),
            pl.BlockSpec((_H, _H), lambda i: (0, 0)),
            pl.BlockSpec((_H, 1), lambda i: (0, 0)),
            pl.BlockSpec((_H, _DIN), lambda i: (0, 0)),
        ],
        out_specs=pl.BlockSpec((_H, _N), lambda i: (0, 0)),
        out_shape=jax.ShapeDtypeStruct((_H, _N), f32),
        scratch_shapes=[
            pltpu.VMEM((_H + 8, _N), f32),
            pltpu.VMEM((_H, _N), jnp.bfloat16),
        ],
        interpret=False,
    )

    ht = jnp.zeros((_H, _N), f32)
    ht1 = step(sigt, srcr, dstc, ht, whh2, bcatt, wcatt)
    ht2 = step(sigt, srcr, dstc, ht1, whh2, bcatt, wcatt)
    ht3 = step(sigt, srcr, dstc, ht2, whh2, bcatt, wcatt)

    houts = jnp.stack([ht3.T, ht2.T])

    rb = 64
    cbt = pl.pallas_call(
        _cbt_kernel,
        grid=(2, _N // rb),
        in_specs=[
            pl.BlockSpec((1, _N, _H), lambda t, i: (t, 0, 0)),
            pl.BlockSpec((1, rb, _H), lambda t, i: (t, i, 0)),
        ],
        out_specs=pl.BlockSpec((1, rb, _N), lambda t, i: (t, i, 0)),
        out_shape=jax.ShapeDtypeStruct((2, _N, _N), f32),
        interpret=False,
    )(houts, houts)

    return cbt


# speedup vs baseline: 19.8226x; 1.0888x over previous
import jax
import jax.numpy as jnp
from jax.experimental import pallas as pl
from jax.experimental.pallas import tpu as pltpu

_N = 512
_DIN = 16
_H = 128
_E = _N * _N
_B = 8192
_NB = _E // _B


def _stats_kernel(ea_ref, stats_ref):
    i = pl.program_id(0)

    @pl.when(i == 0)
    def _init():
        stats_ref[0, 0] = 0.0
        stats_ref[0, 1] = 0.0

    ea = ea_ref[...]
    stats_ref[0, 0] += jnp.sum(ea)
    stats_ref[0, 1] += jnp.sum(ea * ea)


def _sigmoid_kernel(ea_ref, lnw_ref, lnb_ref, ms_ref, out_ref):
    mu = ms_ref[0, 0]
    rstd = ms_ref[0, 1]
    y = (ea_ref[...] - mu) * rstd * lnw_ref[...] + lnb_ref[...]
    out_ref[...] = jax.nn.sigmoid(y)


def _step_kernel(sigt_ref, srcr_ref, dstc_ref, ht_ref, whh2_ref, bcatt_ref,
                 wcatt_ref, htout_ref, acct_ref, gbft_ref):
    i = pl.program_id(0)

    @pl.when(i == 0)
    def _init():
        gt = jnp.dot(whh2_ref[...], ht_ref[...],
                     preferred_element_type=jnp.float32) + bcatt_ref[...]
        gbft_ref[...] = gt.astype(jnp.bfloat16)
        acct_ref[...] = jnp.zeros_like(acct_ref)

    one = jnp.bfloat16(1.0)
    zero = jnp.bfloat16(0.0)

    xt = jnp.dot(wcatt_ref[...], sigt_ref[...],
                 preferred_element_type=jnp.float32)
    src = jnp.broadcast_to(srcr_ref[0], (_N, _B))
    iota_n = jax.lax.broadcasted_iota(jnp.int32, (_N, _B), 0)
    ohst = jnp.where(iota_n == src, 1.0, 0.0).astype(jnp.bfloat16)
    gst = jnp.dot(gbft_ref[...], ohst,
                  preferred_element_type=jnp.float32)
    mt = jnp.tanh(xt + gst)
    maug = jnp.concatenate(
        [mt.astype(jnp.bfloat16), jnp.full((8, _B), one)], axis=0)
    dst = jnp.broadcast_to(dstc_ref[0].astype(jnp.int16), (_B, _N))
    iota_b = jax.lax.broadcasted_iota(jnp.int16, (_B, _N), 1)
    ohdt = jnp.where(iota_b == dst, one, zero)
    acct_ref[...] += jnp.dot(maug, ohdt,
                             preferred_element_type=jnp.float32)

    @pl.when(i == _NB - 1)
    def _fin():
        cnt = acct_ref[_H:_H + 1, :]
        htout_ref[...] = acct_ref[:_H, :] / jnp.maximum(cnt, 1.0)


def _cbt_kernel(ofull_ref, oblk_ref, out_ref):
    o = ofull_ref[0]
    oi = oblk_ref[0]
    jc = min(128, _N)
    for j in range(_N // jc):
        oj = o[j * jc:(j + 1) * jc, :]
        d = jnp.sum(jnp.abs(oi[:, None, :] - oj[None, :, :]), axis=2)
        out_ref[0, :, j * jc:(j + 1) * jc] = d


def kernel(edge_attr, edge_index, ln_w, ln_b, W_ih0, W_hh0, b_ih0, b_hh0,
           W_ih1, W_hh1, b_ih1, b_hh1, W_ih2, W_hh2, b_ih2, b_hh2):
    f32 = jnp.float32
    src = edge_index[0].astype(jnp.int32)
    dst = edge_index[1].astype(jnp.int32)
    srcr = src.reshape(_NB, 1, _B)
    dstc = dst.reshape(_NB, _B, 1)
    eat = edge_attr.T
    lnwt = ln_w.T
    lnbt = ln_b.T

    wcatt = jnp.zeros((_H, _DIN), f32)
    whh2 = jnp.zeros((_H, _H), f32)
    bcatt = jnp.zeros((_H, 1), f32)
    off = 0
    for W_ih, W_hh, b_ih, b_hh in ((W_ih0, W_hh0, b_ih0, b_hh0),
                                   (W_ih1, W_hh1, b_ih1, b_hh1),
                                   (W_ih2, W_hh2, b_ih2, b_hh2)):
        h = W_ih.shape[0]
        wcatt = wcatt.at[off:off + h, :].set(W_ih)
        whh2 = whh2.at[off:off + h, off:off + h].set(W_hh)
        bcatt = bcatt.at[off:off + h, 0].set(b_ih + b_hh)
        off += h

    stats = pl.pallas_call(
        _stats_kernel,
        grid=(_NB,),
        in_specs=[pl.BlockSpec((_DIN, _B), lambda i: (0, i))],
        out_specs=pl.BlockSpec(memory_space=pltpu.SMEM),
        out_shape=jax.ShapeDtypeStruct((1, 2), f32),
        interpret=False,
    )(eat)

    nelem = float(_E * _DIN)
    mu = stats[0, 0] / nelem
    var = stats[0, 1] / nelem - mu * mu
    rstd = jax.lax.rsqrt(var + 1e-5)
    musig = jnp.stack([mu, rstd]).reshape(1, 2)

    sigt = pl.pallas_call(
        _sigmoid_kernel,
        grid=(_NB,),
        in_specs=[
            pl.BlockSpec((_DIN, _B), lambda i: (0, i)),
            pl.BlockSpec((_DIN, _B), lambda i: (0, i)),
            pl.BlockSpec((_DIN, _B), lambda i: (0, i)),
            pl.BlockSpec(memory_space=pltpu.SMEM),
        ],
        out_specs=pl.BlockSpec((_DIN, _B), lambda i: (0, i)),
        out_shape=jax.ShapeDtypeStruct((_DIN, _E), f32),
        interpret=False,
    )(eat, lnwt, lnbt, musig)

    step = pl.pallas_call(
        _step_kernel,
        grid=(_NB,),
        in_specs=[
            pl.BlockSpec((_DIN, _B), lambda i: (0, i)),
            pl.BlockSpec((1, 1, _B), lambda i: (i, 0, 0)),
            pl.BlockSpec((1, _B, 1), lambda i: (i, 0, 0)),
            pl.BlockSpec((_H, _N), lambda i: (0, 0)),
            pl.BlockSpec((_H, _H), lambda i: (0, 0)),
            pl.BlockSpec((_H, 1), lambda i: (0, 0)),
            pl.BlockSpec((_H, _DIN), lambda i: (0, 0)),
        ],
        out_specs=pl.BlockSpec((_H, _N), lambda i: (0, 0)),
        out_shape=jax.ShapeDtypeStruct((_H, _N), f32),
        scratch_shapes=[
            pltpu.VMEM((_H + 8, _N), f32),
            pltpu.VMEM((_H, _N), jnp.bfloat16),
        ],
        interpret=False,
    )

    ht = jnp.zeros((_H, _N), f32)
    ht1 = step(sigt, srcr, dstc, ht, whh2, bcatt, wcatt)
    ht2 = step(sigt, srcr, dstc, ht1, whh2, bcatt, wcatt)
    ht3 = step(sigt, srcr, dstc, ht2, whh2, bcatt, wcatt)

    houts = jnp.stack([ht3.T, ht2.T])

    rb = 64
    cbt = pl.pallas_call(
        _cbt_kernel,
        grid=(2, _N // rb),
        in_specs=[
            pl.BlockSpec((1, _N, _H), lambda t, i: (t, 0, 0)),
            pl.BlockSpec((1, rb, _H), lambda t, i: (t, i, 0)),
        ],
        out_specs=pl.BlockSpec((1, rb, _N), lambda t, i: (t, i, 0)),
        out_shape=jax.ShapeDtypeStruct((2, _N, _N), f32),
        interpret=False,
    )(houts, houts)

    return cbt


# speedup vs baseline: 20.4935x; 1.0338x over previous
import jax
import jax.numpy as jnp
from jax.experimental import pallas as pl
from jax.experimental.pallas import tpu as pltpu

_N = 512
_DIN = 16
_H = 128
_E = _N * _N
_B = 16384
_NB = _E // _B


def _stats_kernel(ea_ref, stats_ref):
    i = pl.program_id(0)

    @pl.when(i == 0)
    def _init():
        stats_ref[0, 0] = 0.0
        stats_ref[0, 1] = 0.0

    ea = ea_ref[...]
    stats_ref[0, 0] += jnp.sum(ea)
    stats_ref[0, 1] += jnp.sum(ea * ea)


def _sigmoid_kernel(ea_ref, lnw_ref, lnb_ref, ms_ref, out_ref):
    mu = ms_ref[0, 0]
    rstd = ms_ref[0, 1]
    y = (ea_ref[...] - mu) * rstd * lnw_ref[...] + lnb_ref[...]
    out_ref[...] = jax.nn.sigmoid(y)


def _step_kernel(sigt_ref, srcr_ref, dstc_ref, ht_ref, whh2_ref, bcatt_ref,
                 wcatt_ref, htout_ref, acct_ref, gbft_ref):
    i = pl.program_id(0)

    @pl.when(i == 0)
    def _init():
        gt = jnp.dot(whh2_ref[...], ht_ref[...],
                     preferred_element_type=jnp.float32) + bcatt_ref[...]
        gbft_ref[...] = gt.astype(jnp.bfloat16)
        acct_ref[...] = jnp.zeros_like(acct_ref)

    one = jnp.bfloat16(1.0)
    zero = jnp.bfloat16(0.0)

    xt = jnp.dot(wcatt_ref[...], sigt_ref[...],
                 preferred_element_type=jnp.float32)
    src = jnp.broadcast_to(srcr_ref[0], (_N, _B))
    iota_n = jax.lax.broadcasted_iota(jnp.int32, (_N, _B), 0)
    ohst = jnp.where(iota_n == src, 1.0, 0.0).astype(jnp.bfloat16)
    gst = jnp.dot(gbft_ref[...], ohst,
                  preferred_element_type=jnp.float32)
    mt = jnp.tanh(xt + gst)
    maug = jnp.concatenate(
        [mt.astype(jnp.bfloat16), jnp.full((8, _B), one)], axis=0)
    dst = jnp.broadcast_to(dstc_ref[0].astype(jnp.int16), (_B, _N))
    iota_b = jax.lax.broadcasted_iota(jnp.int16, (_B, _N), 1)
    ohdt = jnp.where(iota_b == dst, one, zero)
    acct_ref[...] += jnp.dot(maug, ohdt,
                             preferred_element_type=jnp.float32)

    @pl.when(i == _NB - 1)
    def _fin():
        cnt = acct_ref[_H:_H + 1, :]
        htout_ref[...] = acct_ref[:_H, :] / jnp.maximum(cnt, 1.0)


def _cbt_kernel(ofull_ref, oblk_ref, out_ref):
    o = ofull_ref[0]
    oi = oblk_ref[0]
    jc = min(128, _N)
    for j in range(_N // jc):
        oj = o[j * jc:(j + 1) * jc, :]
        d = jnp.sum(jnp.abs(oi[:, None, :] - oj[None, :, :]), axis=2)
        out_ref[0, :, j * jc:(j + 1) * jc] = d


def kernel(edge_attr, edge_index, ln_w, ln_b, W_ih0, W_hh0, b_ih0, b_hh0,
           W_ih1, W_hh1, b_ih1, b_hh1, W_ih2, W_hh2, b_ih2, b_hh2):
    f32 = jnp.float32
    src = edge_index[0].astype(jnp.int32)
    dst = edge_index[1].astype(jnp.int32)
    srcr = src.reshape(_NB, 1, _B)
    dstc = dst.reshape(_NB, _B, 1)
    eat = edge_attr.T
    lnwt = ln_w.T
    lnbt = ln_b.T

    wcatt = jnp.zeros((_H, _DIN), f32)
    whh2 = jnp.zeros((_H, _H), f32)
    bcatt = jnp.zeros((_H, 1), f32)
    off = 0
    for W_ih, W_hh, b_ih, b_hh in ((W_ih0, W_hh0, b_ih0, b_hh0),
                                   (W_ih1, W_hh1, b_ih1, b_hh1),
                                   (W_ih2, W_hh2, b_ih2, b_hh2)):
        h = W_ih.shape[0]
        wcatt = wcatt.at[off:off + h, :].set(W_ih)
        whh2 = whh2.at[off:off + h, off:off + h].set(W_hh)
        bcatt = bcatt.at[off:off + h, 0].set(b_ih + b_hh)
        off += h

    stats = pl.pallas_call(
        _stats_kernel,
        grid=(_NB,),
        in_specs=[pl.BlockSpec((_DIN, _B), lambda i: (0, i))],
        out_specs=pl.BlockSpec(memory_space=pltpu.SMEM),
        out_shape=jax.ShapeDtypeStruct((1, 2), f32),
        interpret=False,
    )(eat)

    nelem = float(_E * _DIN)
    mu = stats[0, 0] / nelem
    var = stats[0, 1] / nelem - mu * mu
    rstd = jax.lax.rsqrt(var + 1e-5)
    musig = jnp.stack([mu, rstd]).reshape(1, 2)

    sigt = pl.pallas_call(
        _sigmoid_kernel,
        grid=(_NB,),
        in_specs=[
            pl.BlockSpec((_DIN, _B), lambda i: (0, i)),
            pl.BlockSpec((_DIN, _B), lambda i: (0, i)),
            pl.BlockSpec((_DIN, _B), lambda i: (0, i)),
            pl.BlockSpec(memory_space=pltpu.SMEM),
        ],
        out_specs=pl.BlockSpec((_DIN, _B), lambda i: (0, i)),
        out_shape=jax.ShapeDtypeStruct((_DIN, _E), f32),
        interpret=False,
    )(eat, lnwt, lnbt, musig)

    step = pl.pallas_call(
        _step_kernel,
        grid=(_NB,),
        in_specs=[
            pl.BlockSpec((_DIN, _B), lambda i: (0, i)),
            pl.BlockSpec((1, 1, _B), lambda i: (i, 0, 0)),
            pl.BlockSpec((1, _B, 1), lambda i: (i, 0, 0)),
            pl.BlockSpec((_H, _N), lambda i: (0, 0)),
            pl.BlockSpec((_H, _H), lambda i: (0, 0)),
            pl.BlockSpec((_H, 1), lambda i: (0, 0)),
            pl.BlockSpec((_H, _DIN), lambda i: (0, 0)),
        ],
        out_specs=pl.BlockSpec((_H, _N), lambda i: (0, 0)),
        out_shape=jax.ShapeDtypeStruct((_H, _N), f32),
        scratch_shapes=[
            pltpu.VMEM((_H + 8, _N), f32),
            pltpu.VMEM((_H, _N), jnp.bfloat16),
        ],
        interpret=False,
    )

    ht = jnp.zeros((_H, _N), f32)
    ht1 = step(sigt, srcr, dstc, ht, whh2, bcatt, wcatt)
    ht2 = step(sigt, srcr, dstc, ht1, whh2, bcatt, wcatt)
    ht3 = step(sigt, srcr, dstc, ht2, whh2, bcatt, wcatt)

    houts = jnp.stack([ht3.T, ht2.T])

    rb = 64
    cbt = pl.pallas_call(
        _cbt_kernel,
        grid=(2, _N // rb),
        in_specs=[
            pl.BlockSpec((1, _N, _H), lambda t, i: (t, 0, 0)),
            pl.BlockSpec((1, rb, _H), lambda t, i: (t, i, 0)),
        ],
        out_specs=pl.BlockSpec((1, rb, _N), lambda t, i: (t, i, 0)),
        out_shape=jax.ShapeDtypeStruct((2, _N, _N), f32),
        interpret=False,
    )(houts, houts)

    return cbt
